# baseline (speedup 1.0000x reference)
import jax
import jax.numpy as jnp
from jax import lax
from jax.experimental import pallas as pl
from jax.experimental.pallas import tpu as pltpu
from jax.experimental.pallas import tpu_sc as plsc

BATCH = 4096
SEQ = 200
DIM = 64
NC, NS, L = 2, 16, 16
NW = NC * NS
BS_W = BATCH // NW
OUT_S = SEQ - 1


def _body(ws_hbm, tab_hbm, out_hbm, idx_v, buf0_v, buf1_v, gsem, wsem0, wsem1):
    wid = lax.axis_index("s") * NC + lax.axis_index("c")
    b0 = wid * BS_W
    pltpu.sync_copy(ws_hbm.at[:, pl.ds(b0, BS_W)], idx_v)
    bufs = (buf0_v, buf1_v)
    wsems = (wsem0, wsem1)

    def drain(k, count):
        for _ in range(count):
            pltpu.make_async_copy(
                bufs[k].at[:, pl.ds(0, DIM)],
                out_hbm.at[pl.ds(0, BS_W), pl.ds(0, DIM)],
                wsems[k],
            ).wait()

    def issue(c, k):
        pltpu.async_copy(tab_hbm.at[idx_v.at[c]], bufs[k], gsem).wait()

        @pl.when(c < OUT_S)
        def _():
            pltpu.async_copy(
                bufs[k].at[:, pl.ds(0, DIM)],
                out_hbm.at[pl.ds(c * BATCH + b0, BS_W), pl.ds(0, DIM)],
                wsems[k],
            )

        @pl.when(c > 0)
        def _():
            pltpu.async_copy(
                bufs[k].at[:, pl.ds(0, DIM)],
                out_hbm.at[pl.ds((c - 1) * BATCH + b0, BS_W), pl.ds(DIM, DIM)],
                wsems[k],
            )

    def step(t, carry):
        @pl.when(t == 1)
        def _():
            drain(0, 1)

        @pl.when(t >= 2)
        def _():
            drain(0, 2)

        issue(2 * t, 0)

        @pl.when(t >= 1)
        def _():
            drain(1, 2)

        issue(2 * t + 1, 1)
        return carry

    lax.fori_loop(0, SEQ // 2, step, 0)
    drain(0, 2)
    drain(1, 1)


@jax.jit
def kernel(whitespace, embed_table):
    ws_t = whitespace.T.astype(jnp.int32)
    tab_p = jnp.pad(embed_table, ((0, 0), (0, 2 * DIM - embed_table.shape[1])))
    mesh = plsc.VectorSubcoreMesh(
        core_axis_name="c", subcore_axis_name="s", num_cores=NC, num_subcores=NS
    )
    out = pl.kernel(
        _body,
        out_type=jax.ShapeDtypeStruct((OUT_S * BATCH, 2 * DIM), jnp.float32),
        mesh=mesh,
        compiler_params=pltpu.CompilerParams(use_tc_tiling_on_sc=False),
        scratch_types=[
            pltpu.VMEM((SEQ, BS_W), jnp.int32),
            pltpu.VMEM((BS_W, 2 * DIM), jnp.float32),
            pltpu.VMEM((BS_W, 2 * DIM), jnp.float32),
            pltpu.SemaphoreType.DMA,
            pltpu.SemaphoreType.DMA,
            pltpu.SemaphoreType.DMA,
        ],
    )(ws_t, tab_p)
    return out.reshape(OUT_S, BATCH, 2 * DIM).transpose(1, 0, 2)


# speedup vs baseline: 1.3078x; 1.3078x over previous
import jax
import jax.numpy as jnp
from jax import lax
from jax.experimental import pallas as pl
from jax.experimental.pallas import tpu as pltpu
from jax.experimental.pallas import tpu_sc as plsc

BATCH = 4096
SEQ = 200
DIM = 64
NC, NS, L = 2, 16, 16
NW = NC * NS
BS_W = BATCH // NW
OUT_S = SEQ - 1
NB = 4


def _body(
    ws_hbm, tab_hbm, out_hbm, idx_v,
    buf0_v, buf1_v, buf2_v, buf3_v,
    gsem0, gsem1, wsem0, wsem1, wsem2, wsem3,
):
    wid = lax.axis_index("s") * NC + lax.axis_index("c")
    b0 = wid * BS_W
    pltpu.sync_copy(ws_hbm.at[:, pl.ds(b0, BS_W)], idx_v)
    bufs = (buf0_v, buf1_v, buf2_v, buf3_v)
    gsems = (gsem0, gsem1)
    wsems = (wsem0, wsem1, wsem2, wsem3)

    def drain_writes(k, count):
        for _ in range(count):
            pltpu.make_async_copy(
                bufs[k], out_hbm.at[pl.ds(0, BS_W), pl.ds(0, DIM)], wsems[k]
            ).wait()

    def start_gather(c, k):
        pltpu.async_copy(tab_hbm.at[idx_v.at[c]], bufs[k], gsems[k % 2])

    def wait_gather(k):
        pltpu.make_async_copy(
            tab_hbm.at[idx_v.at[0]], bufs[k], gsems[k % 2]
        ).wait()

    def write_col(c, k):
        @pl.when(c < OUT_S)
        def _():
            pltpu.async_copy(
                bufs[k],
                out_hbm.at[pl.ds(c * BATCH + b0, BS_W), pl.ds(0, DIM)],
                wsems[k],
            )

        @pl.when(c > 0)
        def _():
            pltpu.async_copy(
                bufs[k],
                out_hbm.at[pl.ds((c - 1) * BATCH + b0, BS_W), pl.ds(DIM, DIM)],
                wsems[k],
            )

    start_gather(0, 0)

    def step(t, carry):
        for k in range(NB):
            c = NB * t + k
            nk = (k + 1) % NB

            @pl.when(c + 1 < SEQ)
            def _():
                @pl.when(c - 3 == 0)
                def _():
                    drain_writes(nk, 1)

                @pl.when(c - 3 >= 1)
                def _():
                    drain_writes(nk, 2)

                start_gather(c + 1, nk)

            wait_gather(k)
            write_col(c, k)
        return carry

    lax.fori_loop(0, SEQ // NB, step, 0)
    drain_writes(0, 2)
    drain_writes(1, 2)
    drain_writes(2, 2)
    drain_writes(3, 1)


@jax.jit
def kernel(whitespace, embed_table):
    ws_t = whitespace.T.astype(jnp.int32)
    mesh = plsc.VectorSubcoreMesh(
        core_axis_name="c", subcore_axis_name="s", num_cores=NC, num_subcores=NS
    )
    out = pl.kernel(
        _body,
        out_type=jax.ShapeDtypeStruct((OUT_S * BATCH, 2 * DIM), jnp.float32),
        mesh=mesh,
        compiler_params=pltpu.CompilerParams(use_tc_tiling_on_sc=False),
        scratch_types=[
            pltpu.VMEM((SEQ, BS_W), jnp.int32),
            pltpu.VMEM((BS_W, DIM), jnp.float32),
            pltpu.VMEM((BS_W, DIM), jnp.float32),
            pltpu.VMEM((BS_W, DIM), jnp.float32),
            pltpu.VMEM((BS_W, DIM), jnp.float32),
            pltpu.SemaphoreType.DMA,
            pltpu.SemaphoreType.DMA,
            pltpu.SemaphoreType.DMA,
            pltpu.SemaphoreType.DMA,
            pltpu.SemaphoreType.DMA,
            pltpu.SemaphoreType.DMA,
        ],
    )(ws_t, embed_table)
    return out.reshape(OUT_S, BATCH, 2 * DIM).transpose(1, 0, 2)


# speedup vs baseline: 1.4121x; 1.0797x over previous
import jax
import jax.numpy as jnp
from jax import lax
from jax.experimental import pallas as pl
from jax.experimental.pallas import tpu as pltpu
from jax.experimental.pallas import tpu_sc as plsc

BATCH = 4096
SEQ = 200
DIM = 64
NC, NS, L = 2, 16, 16
NW = NC * NS
BS_W = BATCH // NW
OUT_S = SEQ - 1
NB = 4


def _body(
    ws_hbm, tab_hbm, out_hbm, idx_v,
    buf0_v, buf1_v, buf2_v, buf3_v,
    gsem0, gsem1, wsem0, wsem1, wsem2, wsem3,
):
    wid = lax.axis_index("s") * NC + lax.axis_index("c")
    b0 = wid * BS_W
    pltpu.sync_copy(ws_hbm.at[:, pl.ds(b0, BS_W)], idx_v)
    bufs = (buf0_v, buf1_v, buf2_v, buf3_v)
    gsems = (gsem0, gsem1)
    wsems = (wsem0, wsem1, wsem2, wsem3)

    def drain_writes(k, count):
        for _ in range(count):
            pltpu.make_async_copy(
                bufs[k], out_hbm.at[pl.ds(0, BS_W), pl.ds(0, DIM)], wsems[k]
            ).wait()

    def start_gather(c, k):
        pltpu.async_copy(tab_hbm.at[idx_v.at[c]], bufs[k], gsems[k % 2])

    def wait_gather(k):
        pltpu.make_async_copy(
            tab_hbm.at[idx_v.at[0]], bufs[k], gsems[k % 2]
        ).wait()

    def write_col(c, k):
        @pl.when(c < OUT_S)
        def _():
            pltpu.async_copy(
                bufs[k],
                out_hbm.at[pl.ds(c * BATCH + b0, BS_W), pl.ds(0, DIM)],
                wsems[k],
            )

        @pl.when(c > 0)
        def _():
            pltpu.async_copy(
                bufs[k],
                out_hbm.at[pl.ds((c - 1) * BATCH + b0, BS_W), pl.ds(DIM, DIM)],
                wsems[k],
            )

    start_gather(0, 0)

    def step(t, carry):
        for k in range(NB):
            c = NB * t + k
            nk = (k + 1) % NB

            @pl.when(c + 1 < SEQ)
            def _():
                @pl.when(c - 3 == 0)
                def _():
                    drain_writes(nk, 1)

                @pl.when(c - 3 >= 1)
                def _():
                    drain_writes(nk, 2)

                start_gather(c + 1, nk)

            wait_gather(k)
            write_col(c, k)
        return carry

    lax.fori_loop(0, SEQ // NB, step, 0)
    drain_writes(0, 2)
    drain_writes(1, 2)
    drain_writes(2, 2)
    drain_writes(3, 1)


@jax.jit
def kernel(whitespace, embed_table):
    ws_t = whitespace.T.astype(jnp.int32) * 2
    tab_p = jnp.concatenate(
        [embed_table, jnp.zeros(embed_table.shape, embed_table.dtype)], axis=1
    ).reshape(2 * embed_table.shape[0], embed_table.shape[1])
    mesh = plsc.VectorSubcoreMesh(
        core_axis_name="c", subcore_axis_name="s", num_cores=NC, num_subcores=NS
    )
    out = pl.kernel(
        _body,
        out_type=jax.ShapeDtypeStruct((OUT_S * BATCH, 2 * DIM), jnp.float32),
        mesh=mesh,
        compiler_params=pltpu.CompilerParams(use_tc_tiling_on_sc=False),
        scratch_types=[
            pltpu.VMEM((SEQ, BS_W), jnp.int32),
            pltpu.VMEM((BS_W, DIM), jnp.float32),
            pltpu.VMEM((BS_W, DIM), jnp.float32),
            pltpu.VMEM((BS_W, DIM), jnp.float32),
            pltpu.VMEM((BS_W, DIM), jnp.float32),
            pltpu.SemaphoreType.DMA,
            pltpu.SemaphoreType.DMA,
            pltpu.SemaphoreType.DMA,
            pltpu.SemaphoreType.DMA,
            pltpu.SemaphoreType.DMA,
            pltpu.SemaphoreType.DMA,
        ],
    )(ws_t, tab_p)
    return out.reshape(OUT_S, BATCH, 2 * DIM).transpose(1, 0, 2)
